# baseline (speedup 1.0000x reference)
import functools

import jax
import jax.numpy as jnp
from jax import lax
from jax.experimental import pallas as pl
from jax.experimental.pallas import tpu as pltpu
from jax.experimental.pallas import tpu_sc as plsc

_L = 16
_NCLS = 1024
_BATCH = 16384
_NSUB = 16
_CHUNK = _BATCH // _NSUB
_VECS = _CHUNK // _L
_CPS = _NCLS // _NSUB

_mesh = plsc.VectorSubcoreMesh(
    core_axis_name="c", subcore_axis_name="s", num_cores=1)


@functools.partial(
    pl.kernel,
    out_type=jax.ShapeDtypeStruct((_NSUB, 2 * _L), jnp.float32),
    mesh=_mesh,
    compiler_params=pltpu.CompilerParams(needs_layout_passes=False),
    scratch_types=[
        pltpu.VMEM((_CHUNK,), jnp.int32),
        pltpu.VMEM((_CHUNK,), jnp.int32),
        pltpu.VMEM((2 * _NCLS,), jnp.float32),
        pltpu.VMEM((_NSUB, _CPS), jnp.float32),
        pltpu.VMEM((_NSUB, _CPS), jnp.float32),
        pltpu.VMEM((2 * _L,), jnp.float32),
        pltpu.SemaphoreType.DMA,
        pltpu.VMEM_SHARED((_NSUB, 2 * _NCLS), jnp.float32),
    ],
)
def _precision_part(yt_hbm, yp_hbm, part_hbm,
                    predv, truev, histv, cbuf, tbuf, redv, dsem, shared_h):
    sid = lax.axis_index("s")
    base = sid * _CHUNK

    in_cp1 = pltpu.async_copy(yp_hbm.at[pl.ds(base, _CHUNK)], predv, dsem)
    in_cp2 = pltpu.async_copy(yt_hbm.at[pl.ds(base, _CHUNK)], truev, dsem)

    zeros = jnp.zeros((_L,), jnp.float32)

    @plsc.parallel_loop(0, 2 * _NCLS // _L, unroll=8)
    def _zero_body(i):
        histv[pl.ds(i * _L, _L)] = zeros

    in_cp1.wait()
    in_cp2.wait()

    @plsc.parallel_loop(0, _VECS, unroll=8)
    def _hist_body(i):
        p = predv[pl.ds(i * _L, _L)]
        t = truev[pl.ds(i * _L, _L)]
        rc_all, last_all = plsc.scan_count(p)
        plsc.addupdate_scatter(
            histv, [p], rc_all.astype(jnp.float32), mask=last_all)
        rc_tp, last_tp = plsc.scan_count(p, mask=p == t)
        plsc.addupdate_scatter(
            histv, [p + _NCLS], rc_tp.astype(jnp.float32), mask=last_tp)

    pltpu.sync_copy(histv, shared_h.at[sid])
    plsc.subcore_barrier()

    col = sid * _CPS
    cps = []
    for r in range(_NSUB):
        cps.append(pltpu.async_copy(
            shared_h.at[r, pl.ds(col, _CPS)], cbuf.at[r], dsem))
        cps.append(pltpu.async_copy(
            shared_h.at[r, pl.ds(_NCLS + col, _CPS)], tbuf.at[r], dsem))
    for cp in cps:
        cp.wait()

    s_acc = jnp.zeros((_L,), jnp.float32)
    n_acc = jnp.zeros((_L,), jnp.float32)
    for j in range(_CPS // _L):
        cnt = jnp.zeros((_L,), jnp.float32)
        tp = jnp.zeros((_L,), jnp.float32)
        for r in range(_NSUB):
            cnt = cnt + cbuf[r, pl.ds(j * _L, _L)]
            tp = tp + tbuf[r, pl.ds(j * _L, _L)]
        valid = cnt > 0.0
        prec = tp / jnp.where(valid, cnt, 1.0)
        s_acc = s_acc + jnp.where(valid, prec, 0.0)
        n_acc = n_acc + jnp.where(valid, 1.0, 0.0)

    redv[pl.ds(0, _L)] = jnp.full((_L,), jnp.sum(s_acc), jnp.float32)
    redv[pl.ds(_L, _L)] = jnp.full((_L,), jnp.sum(n_acc), jnp.float32)
    pltpu.sync_copy(redv, part_hbm.at[sid])


def _combine_body(part_ref, out_ref):
    part = part_ref[...]
    s_tot = jnp.sum(part[:, 0:1])
    n_tot = jnp.sum(part[:, _L:_L + 1])
    out_ref[...] = jnp.full((1, 1), s_tot / n_tot, jnp.float32)


_precision_combine = pl.pallas_call(
    _combine_body,
    out_shape=jax.ShapeDtypeStruct((1, 1), jnp.float32),
)


def kernel(y_true, y_pred):
    part = _precision_part(y_true, y_pred)
    out = _precision_combine(part)
    return out[0, 0]


# speedup vs baseline: 1.0144x; 1.0144x over previous
import functools

import jax
import jax.numpy as jnp
from jax import lax
from jax.experimental import pallas as pl
from jax.experimental.pallas import tpu as pltpu
from jax.experimental.pallas import tpu_sc as plsc

_L = 16
_NCLS = 1024
_BATCH = 16384
_NSUB = 16
_CHUNK = _BATCH // _NSUB
_VECS = _CHUNK // _L
_CPS = _NCLS // _NSUB

_mesh = plsc.VectorSubcoreMesh(
    core_axis_name="c", subcore_axis_name="s", num_cores=1)


@functools.partial(
    pl.kernel,
    out_type=jax.ShapeDtypeStruct((_NSUB, 2 * _L), jnp.float32),
    mesh=_mesh,
    compiler_params=pltpu.CompilerParams(needs_layout_passes=False),
    scratch_types=[
        pltpu.VMEM((_CHUNK,), jnp.int32),
        pltpu.VMEM((_CHUNK,), jnp.int32),
        pltpu.VMEM((2 * _NCLS,), jnp.float32),
        pltpu.VMEM((_NSUB, _CPS), jnp.float32),
        pltpu.VMEM((_NSUB, _CPS), jnp.float32),
        pltpu.VMEM((2 * _L,), jnp.float32),
        pltpu.SemaphoreType.DMA,
        pltpu.VMEM_SHARED((_NSUB, 2 * _NCLS), jnp.float32),
    ],
)
def _precision_part(yt_hbm, yp_hbm, part_hbm,
                    predv, truev, histv, cbuf, tbuf, redv, dsem, shared_h):
    sid = lax.axis_index("s")
    base = sid * _CHUNK

    in_cp1 = pltpu.async_copy(yp_hbm.at[pl.ds(base, _CHUNK)], predv, dsem)
    in_cp2 = pltpu.async_copy(yt_hbm.at[pl.ds(base, _CHUNK)], truev, dsem)

    zeros = jnp.zeros((_L,), jnp.float32)

    @plsc.parallel_loop(0, 2 * _NCLS // _L, unroll=8)
    def _zero_body(i):
        histv[pl.ds(i * _L, _L)] = zeros

    in_cp1.wait()
    in_cp2.wait()

    @plsc.parallel_loop(0, _VECS, unroll=4)
    def _hist_body(i):
        p = predv[pl.ds(i * _L, _L)]
        t = truev[pl.ds(i * _L, _L)]
        rc_all, last_all = plsc.scan_count(p)
        plsc.addupdate_scatter(
            histv, [p], rc_all.astype(jnp.float32), mask=last_all)
        rc_tp, last_tp = plsc.scan_count(p, mask=p == t)
        plsc.addupdate_scatter(
            histv, [p + _NCLS], rc_tp.astype(jnp.float32), mask=last_tp)

    pltpu.sync_copy(histv, shared_h.at[sid])
    plsc.subcore_barrier()

    col = sid * _CPS
    cps = []
    for r in range(_NSUB):
        cps.append(pltpu.async_copy(
            shared_h.at[r, pl.ds(col, _CPS)], cbuf.at[r], dsem))
        cps.append(pltpu.async_copy(
            shared_h.at[r, pl.ds(_NCLS + col, _CPS)], tbuf.at[r], dsem))
    for cp in cps:
        cp.wait()

    s_acc = jnp.zeros((_L,), jnp.float32)
    n_acc = jnp.zeros((_L,), jnp.float32)
    for j in range(_CPS // _L):
        cnt = jnp.zeros((_L,), jnp.float32)
        tp = jnp.zeros((_L,), jnp.float32)
        for r in range(_NSUB):
            cnt = cnt + cbuf[r, pl.ds(j * _L, _L)]
            tp = tp + tbuf[r, pl.ds(j * _L, _L)]
        valid = cnt > 0.0
        prec = tp / jnp.where(valid, cnt, 1.0)
        s_acc = s_acc + jnp.where(valid, prec, 0.0)
        n_acc = n_acc + jnp.where(valid, 1.0, 0.0)

    redv[pl.ds(0, _L)] = jnp.full((_L,), jnp.sum(s_acc), jnp.float32)
    redv[pl.ds(_L, _L)] = jnp.full((_L,), jnp.sum(n_acc), jnp.float32)
    pltpu.sync_copy(redv, part_hbm.at[sid])


def _combine_body(part_ref, out_ref):
    part = part_ref[...]
    s_tot = jnp.sum(part[:, 0:1])
    n_tot = jnp.sum(part[:, _L:_L + 1])
    out_ref[...] = jnp.full((1, 1), s_tot / n_tot, jnp.float32)


_precision_combine = pl.pallas_call(
    _combine_body,
    out_shape=jax.ShapeDtypeStruct((1, 1), jnp.float32),
)


def kernel(y_true, y_pred):
    part = _precision_part(y_true, y_pred)
    out = _precision_combine(part)
    return out[0, 0]
